# Initial kernel scaffold; baseline (speedup 1.0000x reference)
import jax
import jax.numpy as jnp
from jax.experimental import pallas as pl


def kernel(x, edge_index, W1, b1, W2, b2):
    raise NotImplementedError("write your pallas kernel here")


# speedup vs baseline: 6.1719x; 6.1719x over previous
import functools

import jax
import jax.numpy as jnp
from jax import lax
from jax.experimental import pallas as pl
from jax.experimental.pallas import tpu as pltpu
from jax.experimental.pallas import tpu_sc as plsc

N = 10000
E = 160000
D = 256
H = 128
LANES = 16
NC = 2
NS = 16
CH = 128
E_PAD = 163840
ZROWS = 10240
RB = 1000

_mesh = plsc.VectorSubcoreMesh(core_axis_name="c", subcore_axis_name="s")


@functools.partial(
    pl.kernel,
    out_type=jax.ShapeDtypeStruct((NC * ZROWS, LANES), jnp.float32),
    mesh=_mesh,
    scratch_types=[
        pltpu.VMEM_SHARED((ZROWS, LANES), jnp.float32),
        pltpu.VMEM((CH, LANES), jnp.float32),
        pltpu.VMEM((CH, LANES), jnp.float32),
        pltpu.VMEM((CH,), jnp.int32),
    ],
)
def _deg_kernel(dst_hbm, degp_hbm, acc, ones_v, zero_v, didx_v):
    cid = lax.axis_index("c")
    tid = lax.axis_index("s")

    def fill(i, carry):
        ones_v[i, :] = jnp.full((LANES,), 1.0, jnp.float32)
        zero_v[i, :] = jnp.zeros((LANES,), jnp.float32)
        return carry

    lax.fori_loop(0, CH, fill, 0)

    rows_per_tile = ZROWS // NS
    for k in range(rows_per_tile // CH):
        pltpu.sync_copy(zero_v, acc.at[pl.ds(tid * rows_per_tile + k * CH, CH)])
    plsc.subcore_barrier()

    per_worker = E_PAD // (NC * NS)
    ebase = (cid * NS + tid) * per_worker

    def chunk(k, carry):
        pltpu.sync_copy(dst_hbm.at[pl.ds(ebase + k * CH, CH)], didx_v)
        pltpu.sync_copy(ones_v, acc.at[didx_v], add=True)
        return carry

    lax.fori_loop(0, per_worker // CH, chunk, 0)
    plsc.subcore_barrier()

    out_rows = ZROWS // NS
    pltpu.sync_copy(
        acc.at[pl.ds(tid * out_rows, out_rows)],
        degp_hbm.at[pl.ds(cid * ZROWS + tid * out_rows, out_rows)],
    )


@functools.partial(
    pl.kernel,
    out_type=jax.ShapeDtypeStruct((NC * ZROWS, H), jnp.float32),
    mesh=_mesh,
    scratch_types=[
        pltpu.VMEM_SHARED((ZROWS, H), jnp.float32),
        pltpu.VMEM((CH, H), jnp.float32),
        pltpu.VMEM((CH,), jnp.int32),
        pltpu.VMEM((CH,), jnp.int32),
        pltpu.SemaphoreType.DMA,
    ],
)
def _agg_kernel(y_hbm, srcs_hbm, dst_hbm, z_hbm, acc, rows_v, sidx_v, didx_v, sem):
    cid = lax.axis_index("c")
    tid = lax.axis_index("s")

    def zfill(i, carry):
        for j in range(H // LANES):
            rows_v[i, pl.ds(j * LANES, LANES)] = jnp.zeros((LANES,), jnp.float32)
        return carry

    lax.fori_loop(0, CH, zfill, 0)

    rows_per_tile = ZROWS // NS
    for k in range(rows_per_tile // CH):
        pltpu.sync_copy(rows_v, acc.at[pl.ds(tid * rows_per_tile + k * CH, CH)])
    plsc.subcore_barrier()

    per_tile = E_PAD // NS
    ebase = tid * per_tile
    sbase = cid * E_PAD + ebase

    def chunk(k, carry):
        pltpu.sync_copy(srcs_hbm.at[pl.ds(sbase + k * CH, CH)], sidx_v)
        pltpu.sync_copy(dst_hbm.at[pl.ds(ebase + k * CH, CH)], didx_v)
        pltpu.async_copy(y_hbm.at[sidx_v], rows_v, sem).wait()
        pltpu.sync_copy(rows_v, acc.at[didx_v], add=True)
        return carry

    lax.fori_loop(0, per_tile // CH, chunk, 0)
    plsc.subcore_barrier()

    out_rows = ZROWS // NS
    pltpu.sync_copy(
        acc.at[pl.ds(tid * out_rows, out_rows)],
        z_hbm.at[pl.ds(cid * ZROWS + tid * out_rows, out_rows)],
    )


def _dinv_block(dp):
    deg = dp[0, :, 0] + dp[1, :, 0] + 1.0
    return lax.rsqrt(deg)


def _mm1_body(x_ref, w_ref, dp_ref, y_ref):
    dinv = _dinv_block(dp_ref[...])
    xw = jnp.dot(x_ref[...], w_ref[...], preferred_element_type=jnp.float32)
    y = xw * dinv[:, None]
    y_ref[0] = y[:, :H]
    y_ref[1] = y[:, H:]


def _mm2_body(z_ref, y_ref, dp_ref, b_ref, w_ref, y2_ref):
    dinv = _dinv_block(dp_ref[...])[:, None]
    h0 = jnp.maximum(dinv * (z_ref[0] + y_ref[0]) + b_ref[0, :H], 0.0)
    h1 = jnp.maximum(dinv * (z_ref[1] + y_ref[1]) + b_ref[0, H:], 0.0)
    h = jnp.concatenate([h0, h1], axis=1)
    xw = jnp.dot(h, w_ref[...], preferred_element_type=jnp.float32)
    y2 = xw * dinv
    y2_ref[0] = y2[:, :H]
    y2_ref[1] = y2[:, H:]


def _epi_body(z_ref, y_ref, dp_ref, b_ref, out_ref):
    dinv = _dinv_block(dp_ref[...])[:, None]
    out_ref[:, :H] = jnp.maximum(dinv * (z_ref[0] + y_ref[0]) + b_ref[0, :H], 0.0)
    out_ref[:, H:] = jnp.maximum(dinv * (z_ref[1] + y_ref[1]) + b_ref[0, H:], 0.0)


_halves_spec = pl.BlockSpec((2, RB, H), lambda i: (0, i, 0))
_dp_spec = pl.BlockSpec((2, RB, LANES), lambda i: (0, i, 0))
_w_spec = pl.BlockSpec((D, D), lambda i: (0, 0))
_b_spec = pl.BlockSpec((1, D), lambda i: (0, 0))

_mm1 = pl.pallas_call(
    _mm1_body,
    grid=(N // RB,),
    in_specs=[pl.BlockSpec((RB, D), lambda i: (i, 0)), _w_spec, _dp_spec],
    out_specs=_halves_spec,
    out_shape=jax.ShapeDtypeStruct((2, N, H), jnp.float32),
)

_mm2 = pl.pallas_call(
    _mm2_body,
    grid=(N // RB,),
    in_specs=[_halves_spec, _halves_spec, _dp_spec, _b_spec, _w_spec],
    out_specs=_halves_spec,
    out_shape=jax.ShapeDtypeStruct((2, N, H), jnp.float32),
)

_epi = pl.pallas_call(
    _epi_body,
    grid=(N // RB,),
    in_specs=[_halves_spec, _halves_spec, _dp_spec, _b_spec],
    out_specs=pl.BlockSpec((RB, D), lambda i: (i, 0)),
    out_shape=jax.ShapeDtypeStruct((N, D), jnp.float32),
)


def kernel(x, edge_index, W1, b1, W2, b2):
    ei = edge_index.astype(jnp.int32)
    pad = E_PAD - E
    src_p = jnp.concatenate([ei[0], jnp.zeros((pad,), jnp.int32)])
    dst_p = jnp.concatenate([ei[1], jnp.full((pad,), N, jnp.int32)])
    srcs2 = jnp.concatenate([src_p, src_p + N])

    degp = _deg_kernel(dst_p).reshape(NC, ZROWS, LANES)
    b1r = b1.reshape(1, D)
    b2r = b2.reshape(1, D)

    y1 = _mm1(x, W1, degp)
    z1 = _agg_kernel(y1.reshape(NC * N, H), srcs2, dst_p).reshape(NC, ZROWS, H)
    y2 = _mm2(z1, y1, degp, b1r, W2)
    z2 = _agg_kernel(y2.reshape(NC * N, H), srcs2, dst_p).reshape(NC, ZROWS, H)
    return _epi(z2, y2, degp, b2r)
